# Initial kernel scaffold; baseline (speedup 1.0000x reference)
import jax
import jax.numpy as jnp
from jax.experimental import pallas as pl


def kernel(x, edge_index, W1, b1, W2, b2, W3, b3, W4, b4, W5, b5):
    raise NotImplementedError("write your pallas kernel here")


# speedup vs baseline: 18.3143x; 18.3143x over previous
import functools

import jax
import jax.numpy as jnp
from jax import lax
from jax.experimental import pallas as pl
from jax.experimental.pallas import tpu as pltpu
from jax.experimental.pallas import tpu_sc as plsc

N = 10000
E = 320000
DF = 128
DH = 64

NC = 2
NS = 16
NW = NC * NS
EPT = E // NW
CHUNK = 125
NCH = EPT // CHUNK
NPAD = 10240
RPS = NPAD // NS

_mesh = plsc.VectorSubcoreMesh(core_axis_name="c", subcore_axis_name="s")
_sc_params = pltpu.CompilerParams(use_tc_tiling_on_sc=False)


@functools.partial(
    pl.kernel,
    mesh=_mesh,
    out_type=jax.ShapeDtypeStruct((NC, NPAD, 8), jnp.float32),
    scratch_types=[
        pltpu.VMEM((NCH, CHUNK), jnp.int32),
        pltpu.VMEM((CHUNK, 8), jnp.float32),
        pltpu.VMEM_SHARED((NPAD, 8), jnp.float32),
    ],
    compiler_params=_sc_params,
)
def _deg_sc(dst_hbm, ones_hbm, zeros_hbm, out_hbm, idx_v, ones_v, acc_sh):
    c = lax.axis_index("c")
    s = lax.axis_index("s")
    wid = c * NS + s
    pltpu.sync_copy(dst_hbm.at[wid], idx_v)
    pltpu.sync_copy(ones_hbm, ones_v)
    base = pl.multiple_of(s * RPS, 8)
    pltpu.sync_copy(zeros_hbm.at[pl.ds(base, RPS)],
                    acc_sh.at[pl.ds(base, RPS)])
    plsc.subcore_barrier()

    def body(j, carry):
        pltpu.sync_copy(ones_v, acc_sh.at[idx_v.at[j]], add=True)
        return carry

    lax.fori_loop(0, NCH, body, 0)
    plsc.subcore_barrier()
    pltpu.sync_copy(acc_sh.at[pl.ds(base, RPS)],
                    out_hbm.at[c].at[pl.ds(base, RPS)])


def _make_agg(D):
    @functools.partial(
        pl.kernel,
        mesh=_mesh,
        out_type=jax.ShapeDtypeStruct((NC, NPAD, D), jnp.float32),
        scratch_types=[
            pltpu.VMEM((NCH, CHUNK), jnp.int32),
            pltpu.VMEM((NCH, CHUNK), jnp.int32),
            pltpu.VMEM((CHUNK, D), jnp.float32),
            pltpu.VMEM_SHARED((NPAD, D), jnp.float32),
            pltpu.SemaphoreType.DMA,
        ],
        compiler_params=_sc_params,
    )
    def agg(src_hbm, dst_hbm, g_hbm, zeros_hbm, out_hbm,
            src_v, dst_v, rows_v, acc_sh, sem):
        c = lax.axis_index("c")
        s = lax.axis_index("s")
        wid = c * NS + s
        pltpu.sync_copy(src_hbm.at[wid], src_v)
        pltpu.sync_copy(dst_hbm.at[wid], dst_v)
        base = pl.multiple_of(s * RPS, 8)
        pltpu.sync_copy(zeros_hbm.at[pl.ds(base, RPS)],
                        acc_sh.at[pl.ds(base, RPS)])
        plsc.subcore_barrier()

        def body(j, carry):
            pltpu.async_copy(g_hbm.at[src_v.at[j]], rows_v, sem).wait()
            pltpu.sync_copy(rows_v, acc_sh.at[dst_v.at[j]], add=True)
            return carry

        lax.fori_loop(0, NCH, body, 0)
        plsc.subcore_barrier()
        pltpu.sync_copy(acc_sh.at[pl.ds(base, RPS)],
                        out_hbm.at[c].at[pl.ds(base, RPS)])

    return agg


_agg64 = _make_agg(DH)
_agg128 = _make_agg(DF)


_BLK = 2000
_MM = (((1,), (0,)), ((), ()))


def _dinv_of(dp_ref):
    deg = dp_ref[0, :, 0:1] + dp_ref[1, :, 0:1] + 1.0
    return lax.rsqrt(deg)


def _stage_a(x, W1, degP):
    def body(x_ref, w_ref, dp_ref, o_ref):
        dinv = _dinv_of(dp_ref)
        t = lax.dot_general(x_ref[...], w_ref[...], _MM,
                            preferred_element_type=jnp.float32)
        o_ref[...] = dinv * t

    return pl.pallas_call(
        body,
        grid=(N // _BLK,),
        in_specs=[
            pl.BlockSpec((_BLK, DF), lambda i: (i, 0)),
            pl.BlockSpec((DF, DH), lambda i: (0, 0)),
            pl.BlockSpec((NC, _BLK, 8), lambda i: (0, i, 0)),
        ],
        out_specs=pl.BlockSpec((_BLK, DH), lambda i: (i, 0)),
        out_shape=jax.ShapeDtypeStruct((N, DH), jnp.float32),
    )(x, W1, degP)


def _stage_b(P, g, degP, b, W):
    def body(p_ref, g_ref, dp_ref, b_ref, w_ref, o_ref):
        dinv = _dinv_of(dp_ref)
        h = jnp.maximum(dinv * (p_ref[0] + p_ref[1] + g_ref[...]) + b_ref[...],
                        0.0)
        o_ref[...] = dinv * lax.dot_general(
            h, w_ref[...], _MM, preferred_element_type=jnp.float32)

    return pl.pallas_call(
        body,
        grid=(N // _BLK,),
        in_specs=[
            pl.BlockSpec((NC, _BLK, DH), lambda i: (0, i, 0)),
            pl.BlockSpec((_BLK, DH), lambda i: (i, 0)),
            pl.BlockSpec((NC, _BLK, 8), lambda i: (0, i, 0)),
            pl.BlockSpec((1, DH), lambda i: (0, 0)),
            pl.BlockSpec((DH, DH), lambda i: (0, 0)),
        ],
        out_specs=pl.BlockSpec((_BLK, DH), lambda i: (i, 0)),
        out_shape=jax.ShapeDtypeStruct((N, DH), jnp.float32),
    )(P, g, degP, b, W)


def _stage_c(P, g, degP, b, W3, W5):
    def body(p_ref, g_ref, dp_ref, b_ref, w3_ref, w5_ref, o3_ref, o5_ref):
        dinv = _dinv_of(dp_ref)
        h = jnp.maximum(dinv * (p_ref[0] + p_ref[1] + g_ref[...]) + b_ref[...],
                        0.0)
        o3_ref[...] = dinv * lax.dot_general(
            h, w3_ref[...], _MM, preferred_element_type=jnp.float32)
        o5_ref[...] = dinv * lax.dot_general(
            h, w5_ref[...], _MM, preferred_element_type=jnp.float32)

    return pl.pallas_call(
        body,
        grid=(N // _BLK,),
        in_specs=[
            pl.BlockSpec((NC, _BLK, DH), lambda i: (0, i, 0)),
            pl.BlockSpec((_BLK, DH), lambda i: (i, 0)),
            pl.BlockSpec((NC, _BLK, 8), lambda i: (0, i, 0)),
            pl.BlockSpec((1, DH), lambda i: (0, 0)),
            pl.BlockSpec((DH, DH), lambda i: (0, 0)),
            pl.BlockSpec((DH, DH), lambda i: (0, 0)),
        ],
        out_specs=[
            pl.BlockSpec((_BLK, DH), lambda i: (i, 0)),
            pl.BlockSpec((_BLK, DH), lambda i: (i, 0)),
        ],
        out_shape=[
            jax.ShapeDtypeStruct((N, DH), jnp.float32),
            jax.ShapeDtypeStruct((N, DH), jnp.float32),
        ],
    )(P, g, degP, b, W3, W5)


def _stage_d(P, g3, degP, b, g5):
    def body(p_ref, g3_ref, dp_ref, b_ref, g5_ref, o_ref):
        dinv = _dinv_of(dp_ref)
        xa = jnp.maximum(
            dinv * (p_ref[0] + p_ref[1] + g3_ref[...]) + b_ref[...], 0.0)
        o_ref[:, :DH] = dinv * xa
        o_ref[:, DH:] = g5_ref[...]

    return pl.pallas_call(
        body,
        grid=(N // _BLK,),
        in_specs=[
            pl.BlockSpec((NC, _BLK, DH), lambda i: (0, i, 0)),
            pl.BlockSpec((_BLK, DH), lambda i: (i, 0)),
            pl.BlockSpec((NC, _BLK, 8), lambda i: (0, i, 0)),
            pl.BlockSpec((1, DH), lambda i: (0, 0)),
            pl.BlockSpec((_BLK, DH), lambda i: (i, 0)),
        ],
        out_specs=pl.BlockSpec((_BLK, DF), lambda i: (i, 0)),
        out_shape=jax.ShapeDtypeStruct((N, DF), jnp.float32),
    )(P, g3, degP, b, g5)


def _stage_e(P, g, degP, W4, b4, b5):
    def body(p_ref, g_ref, dp_ref, w4_ref, b4_ref, b5_ref, xh_ref, s_ref):
        dinv = _dinv_of(dp_ref)
        comb = dinv * (p_ref[0] + p_ref[1] + g_ref[...])
        a4 = comb[:, :DH]
        xh_ref[...] = jnp.maximum(
            lax.dot_general(a4, w4_ref[...], _MM,
                            preferred_element_type=jnp.float32) + b4_ref[...],
            0.0)
        s_ref[...] = jnp.maximum(comb[:, DH:] + b5_ref[...], 0.0)

    return pl.pallas_call(
        body,
        grid=(N // _BLK,),
        in_specs=[
            pl.BlockSpec((NC, _BLK, DF), lambda i: (0, i, 0)),
            pl.BlockSpec((_BLK, DF), lambda i: (i, 0)),
            pl.BlockSpec((NC, _BLK, 8), lambda i: (0, i, 0)),
            pl.BlockSpec((DH, DF), lambda i: (0, 0)),
            pl.BlockSpec((1, DF), lambda i: (0, 0)),
            pl.BlockSpec((1, DH), lambda i: (0, 0)),
        ],
        out_specs=[
            pl.BlockSpec((_BLK, DF), lambda i: (i, 0)),
            pl.BlockSpec((_BLK, DH), lambda i: (i, 0)),
        ],
        out_shape=[
            jax.ShapeDtypeStruct((N, DF), jnp.float32),
            jax.ShapeDtypeStruct((N, DH), jnp.float32),
        ],
    )(P, g, degP, W4, b4, b5)


def _stage_f(s):
    BM, BN = 1000, 1024
    _TT = (((1,), (1,)), ((), ()))

    def body(a_ref, b_ref, o_ref):
        o_ref[...] = lax.dot_general(a_ref[...], b_ref[...], _TT,
                                     preferred_element_type=jnp.float32)

    return pl.pallas_call(
        body,
        grid=(N // BM, pl.cdiv(N, BN)),
        in_specs=[
            pl.BlockSpec((BM, DH), lambda i, j: (i, 0)),
            pl.BlockSpec((BN, DH), lambda i, j: (j, 0)),
        ],
        out_specs=pl.BlockSpec((BM, BN), lambda i, j: (i, j)),
        out_shape=jax.ShapeDtypeStruct((N, N), jnp.float32),
        compiler_params=pltpu.CompilerParams(
            dimension_semantics=("parallel", "parallel")),
    )(s, s)


def kernel(x, edge_index, W1, b1, W2, b2, W3, b3, W4, b4, W5, b5):
    src = edge_index[0].astype(jnp.int32).reshape(NW, NCH, CHUNK)
    dst = edge_index[1].astype(jnp.int32).reshape(NW, NCH, CHUNK)
    ones8 = jnp.ones((CHUNK, 8), jnp.float32)
    z8 = jnp.zeros((NPAD, 8), jnp.float32)
    z64 = jnp.zeros((NPAD, DH), jnp.float32)
    z128 = jnp.zeros((NPAD, DF), jnp.float32)

    degP = _deg_sc(dst, ones8, z8)
    g1 = _stage_a(x, W1, degP)
    P1 = _agg64(src, dst, g1, z64)
    g2 = _stage_b(P1, g1, degP, b1.reshape(1, -1), W2)
    P2 = _agg64(src, dst, g2, z64)
    g3, g5 = _stage_c(P2, g2, degP, b2.reshape(1, -1), W3, W5)
    P3 = _agg64(src, dst, g3, z64)
    g45 = _stage_d(P3, g3, degP, b3.reshape(1, -1), g5)
    P45 = _agg128(src, dst, g45, z128)
    X_hat, s = _stage_e(P45, g45, degP, W4, b4.reshape(1, -1),
                        b5.reshape(1, -1))
    A_hat = _stage_f(s)
    return (A_hat, X_hat)


# speedup vs baseline: 23.5573x; 1.2863x over previous
import functools

import jax
import jax.numpy as jnp
from jax import lax
from jax.experimental import pallas as pl
from jax.experimental.pallas import tpu as pltpu
from jax.experimental.pallas import tpu_sc as plsc

N = 10000
E = 320000
DF = 128
DH = 64

NC = 2
NS = 16
NW = NC * NS
EPT = E // NW
CHUNK = 125
NCH = EPT // CHUNK
NPAD = 10240
RPS = NPAD // NS

_mesh = plsc.VectorSubcoreMesh(core_axis_name="c", subcore_axis_name="s")
_sc_params = pltpu.CompilerParams(use_tc_tiling_on_sc=False)


@functools.partial(
    pl.kernel,
    mesh=_mesh,
    out_type=jax.ShapeDtypeStruct((NC, NPAD, 8), jnp.float32),
    scratch_types=[
        pltpu.VMEM((NCH, CHUNK), jnp.int32),
        pltpu.VMEM((CHUNK, 8), jnp.float32),
        pltpu.VMEM_SHARED((NPAD, 8), jnp.float32),
    ],
    compiler_params=_sc_params,
)
def _deg_sc(dst_hbm, ones_hbm, zeros_hbm, out_hbm, idx_v, ones_v, acc_sh):
    c = lax.axis_index("c")
    s = lax.axis_index("s")
    wid = c * NS + s
    pltpu.sync_copy(dst_hbm.at[wid], idx_v)
    pltpu.sync_copy(ones_hbm, ones_v)
    base = pl.multiple_of(s * RPS, 8)
    pltpu.sync_copy(zeros_hbm.at[pl.ds(base, RPS)],
                    acc_sh.at[pl.ds(base, RPS)])
    plsc.subcore_barrier()

    def body(j, carry):
        pltpu.sync_copy(ones_v, acc_sh.at[idx_v.at[j]], add=True)
        return carry

    lax.fori_loop(0, NCH, body, 0)
    plsc.subcore_barrier()
    pltpu.sync_copy(acc_sh.at[pl.ds(base, RPS)],
                    out_hbm.at[c].at[pl.ds(base, RPS)])


def _make_agg(D):
    @functools.partial(
        pl.kernel,
        mesh=_mesh,
        out_type=jax.ShapeDtypeStruct((NC, NPAD, D), jnp.float32),
        scratch_types=[
            pltpu.VMEM((NCH, CHUNK), jnp.int32),
            pltpu.VMEM((NCH, CHUNK), jnp.int32),
            pltpu.VMEM((CHUNK, D), jnp.float32),
            pltpu.VMEM((CHUNK, D), jnp.float32),
            pltpu.VMEM_SHARED((NPAD, D), jnp.float32),
            pltpu.SemaphoreType.DMA,
            pltpu.SemaphoreType.DMA,
        ],
        compiler_params=_sc_params,
    )
    def agg(src_hbm, dst_hbm, g_hbm, zeros_hbm, out_hbm,
            src_v, dst_v, rows0_v, rows1_v, acc_sh, sem0, sem1):
        c = lax.axis_index("c")
        s = lax.axis_index("s")
        wid = c * NS + s
        pltpu.sync_copy(src_hbm.at[wid], src_v)
        pltpu.sync_copy(dst_hbm.at[wid], dst_v)
        pltpu.async_copy(g_hbm.at[src_v.at[0]], rows0_v, sem0)
        base = pl.multiple_of(s * RPS, 8)
        pltpu.sync_copy(zeros_hbm.at[pl.ds(base, RPS)],
                        acc_sh.at[pl.ds(base, RPS)])
        plsc.subcore_barrier()

        def body(h, carry):
            j = h * 2
            pltpu.async_copy(g_hbm.at[src_v.at[j + 1]], rows1_v, sem1)
            pltpu.make_async_copy(g_hbm.at[src_v.at[j]], rows0_v, sem0).wait()
            pltpu.sync_copy(rows0_v, acc_sh.at[dst_v.at[j]], add=True)

            @pl.when(j + 2 < NCH)
            def _():
                pltpu.async_copy(g_hbm.at[src_v.at[j + 2]], rows0_v, sem0)

            pltpu.make_async_copy(g_hbm.at[src_v.at[j + 1]], rows1_v,
                                  sem1).wait()
            pltpu.sync_copy(rows1_v, acc_sh.at[dst_v.at[j + 1]], add=True)
            return carry

        lax.fori_loop(0, NCH // 2, body, 0)
        plsc.subcore_barrier()
        pltpu.sync_copy(acc_sh.at[pl.ds(base, RPS)],
                        out_hbm.at[c].at[pl.ds(base, RPS)])

    return agg


_agg64 = _make_agg(DH)


_BLK = 2000
_MM = (((1,), (0,)), ((), ()))


def _dinv_of(dp_ref):
    deg = dp_ref[0, :, 0:1] + dp_ref[1, :, 0:1] + 1.0
    return lax.rsqrt(deg)


def _stage_a(x, W1, degP):
    def body(x_ref, w_ref, dp_ref, o_ref):
        dinv = _dinv_of(dp_ref)
        t = lax.dot_general(x_ref[...], w_ref[...], _MM,
                            preferred_element_type=jnp.float32)
        o_ref[...] = dinv * t

    return pl.pallas_call(
        body,
        grid=(N // _BLK,),
        in_specs=[
            pl.BlockSpec((_BLK, DF), lambda i: (i, 0)),
            pl.BlockSpec((DF, DH), lambda i: (0, 0)),
            pl.BlockSpec((NC, _BLK, 8), lambda i: (0, i, 0)),
        ],
        out_specs=pl.BlockSpec((_BLK, DH), lambda i: (i, 0)),
        out_shape=jax.ShapeDtypeStruct((N, DH), jnp.float32),
    )(x, W1, degP)


def _stage_b(P, g, degP, b, W):
    def body(p_ref, g_ref, dp_ref, b_ref, w_ref, o_ref):
        dinv = _dinv_of(dp_ref)
        h = jnp.maximum(dinv * (p_ref[0] + p_ref[1] + g_ref[...]) + b_ref[...],
                        0.0)
        o_ref[...] = dinv * lax.dot_general(
            h, w_ref[...], _MM, preferred_element_type=jnp.float32)

    return pl.pallas_call(
        body,
        grid=(N // _BLK,),
        in_specs=[
            pl.BlockSpec((NC, _BLK, DH), lambda i: (0, i, 0)),
            pl.BlockSpec((_BLK, DH), lambda i: (i, 0)),
            pl.BlockSpec((NC, _BLK, 8), lambda i: (0, i, 0)),
            pl.BlockSpec((1, DH), lambda i: (0, 0)),
            pl.BlockSpec((DH, DH), lambda i: (0, 0)),
        ],
        out_specs=pl.BlockSpec((_BLK, DH), lambda i: (i, 0)),
        out_shape=jax.ShapeDtypeStruct((N, DH), jnp.float32),
    )(P, g, degP, b, W)


def _stage_c(P, g, degP, b, W3, W5):
    def body(p_ref, g_ref, dp_ref, b_ref, w3_ref, w5_ref, o3_ref, o5_ref):
        dinv = _dinv_of(dp_ref)
        h = jnp.maximum(dinv * (p_ref[0] + p_ref[1] + g_ref[...]) + b_ref[...],
                        0.0)
        o3_ref[...] = dinv * lax.dot_general(
            h, w3_ref[...], _MM, preferred_element_type=jnp.float32)
        o5_ref[...] = dinv * lax.dot_general(
            h, w5_ref[...], _MM, preferred_element_type=jnp.float32)

    return pl.pallas_call(
        body,
        grid=(N // _BLK,),
        in_specs=[
            pl.BlockSpec((NC, _BLK, DH), lambda i: (0, i, 0)),
            pl.BlockSpec((_BLK, DH), lambda i: (i, 0)),
            pl.BlockSpec((NC, _BLK, 8), lambda i: (0, i, 0)),
            pl.BlockSpec((1, DH), lambda i: (0, 0)),
            pl.BlockSpec((DH, DH), lambda i: (0, 0)),
            pl.BlockSpec((DH, DH), lambda i: (0, 0)),
        ],
        out_specs=[
            pl.BlockSpec((_BLK, DH), lambda i: (i, 0)),
            pl.BlockSpec((_BLK, DH), lambda i: (i, 0)),
        ],
        out_shape=[
            jax.ShapeDtypeStruct((N, DH), jnp.float32),
            jax.ShapeDtypeStruct((N, DH), jnp.float32),
        ],
    )(P, g, degP, b, W3, W5)


def _stage_d(P, g3, degP, b):
    def body(p_ref, g3_ref, dp_ref, b_ref, o_ref):
        dinv = _dinv_of(dp_ref)
        xa = jnp.maximum(
            dinv * (p_ref[0] + p_ref[1] + g3_ref[...]) + b_ref[...], 0.0)
        o_ref[...] = dinv * xa

    return pl.pallas_call(
        body,
        grid=(N // _BLK,),
        in_specs=[
            pl.BlockSpec((NC, _BLK, DH), lambda i: (0, i, 0)),
            pl.BlockSpec((_BLK, DH), lambda i: (i, 0)),
            pl.BlockSpec((NC, _BLK, 8), lambda i: (0, i, 0)),
            pl.BlockSpec((1, DH), lambda i: (0, 0)),
        ],
        out_specs=pl.BlockSpec((_BLK, DH), lambda i: (i, 0)),
        out_shape=jax.ShapeDtypeStruct((N, DH), jnp.float32),
    )(P, g3, degP, b)


def _stage_e(P4, g4, P5, g5, degP, W4, b4, b5):
    def body(p4_ref, g4_ref, p5_ref, g5_ref, dp_ref, w4_ref, b4_ref, b5_ref,
             xh_ref, s_ref):
        dinv = _dinv_of(dp_ref)
        a4 = dinv * (p4_ref[0] + p4_ref[1] + g4_ref[...])
        xh_ref[...] = jnp.maximum(
            lax.dot_general(a4, w4_ref[...], _MM,
                            preferred_element_type=jnp.float32) + b4_ref[...],
            0.0)
        s_ref[...] = jnp.maximum(
            dinv * (p5_ref[0] + p5_ref[1] + g5_ref[...]) + b5_ref[...], 0.0)

    return pl.pallas_call(
        body,
        grid=(N // _BLK,),
        in_specs=[
            pl.BlockSpec((NC, _BLK, DH), lambda i: (0, i, 0)),
            pl.BlockSpec((_BLK, DH), lambda i: (i, 0)),
            pl.BlockSpec((NC, _BLK, DH), lambda i: (0, i, 0)),
            pl.BlockSpec((_BLK, DH), lambda i: (i, 0)),
            pl.BlockSpec((NC, _BLK, 8), lambda i: (0, i, 0)),
            pl.BlockSpec((DH, DF), lambda i: (0, 0)),
            pl.BlockSpec((1, DF), lambda i: (0, 0)),
            pl.BlockSpec((1, DH), lambda i: (0, 0)),
        ],
        out_specs=[
            pl.BlockSpec((_BLK, DF), lambda i: (i, 0)),
            pl.BlockSpec((_BLK, DH), lambda i: (i, 0)),
        ],
        out_shape=[
            jax.ShapeDtypeStruct((N, DF), jnp.float32),
            jax.ShapeDtypeStruct((N, DH), jnp.float32),
        ],
    )(P4, g4, P5, g5, degP, W4, b4, b5)


def _stage_f(s):
    BM, BN = 1000, 1024
    _TT = (((1,), (1,)), ((), ()))

    def body(a_ref, b_ref, o_ref):
        o_ref[...] = lax.dot_general(a_ref[...], b_ref[...], _TT,
                                     preferred_element_type=jnp.float32)

    return pl.pallas_call(
        body,
        grid=(N // BM, pl.cdiv(N, BN)),
        in_specs=[
            pl.BlockSpec((BM, DH), lambda i, j: (i, 0)),
            pl.BlockSpec((BN, DH), lambda i, j: (j, 0)),
        ],
        out_specs=pl.BlockSpec((BM, BN), lambda i, j: (i, j)),
        out_shape=jax.ShapeDtypeStruct((N, N), jnp.float32),
        compiler_params=pltpu.CompilerParams(
            dimension_semantics=("parallel", "parallel")),
    )(s, s)


def kernel(x, edge_index, W1, b1, W2, b2, W3, b3, W4, b4, W5, b5):
    src = edge_index[0].astype(jnp.int32).reshape(NW, NCH, CHUNK)
    dst = edge_index[1].astype(jnp.int32).reshape(NW, NCH, CHUNK)
    ones8 = jnp.ones((CHUNK, 8), jnp.float32)
    z8 = jnp.zeros((NPAD, 8), jnp.float32)
    z64 = jnp.zeros((NPAD, DH), jnp.float32)

    degP = _deg_sc(dst, ones8, z8)
    g1 = _stage_a(x, W1, degP)
    P1 = _agg64(src, dst, g1, z64)
    g2 = _stage_b(P1, g1, degP, b1.reshape(1, -1), W2)
    P2 = _agg64(src, dst, g2, z64)
    g3, g5 = _stage_c(P2, g2, degP, b2.reshape(1, -1), W3, W5)
    P3 = _agg64(src, dst, g3, z64)
    g4 = _stage_d(P3, g3, degP, b3.reshape(1, -1))
    P4 = _agg64(src, dst, g4, z64)
    P5 = _agg64(src, dst, g5, z64)
    X_hat, s = _stage_e(P4, g4, P5, g5, degP, W4, b4.reshape(1, -1),
                        b5.reshape(1, -1))
    A_hat = _stage_f(s)
    return (A_hat, X_hat)


# speedup vs baseline: 26.8291x; 1.1389x over previous
import functools

import jax
import jax.numpy as jnp
from jax import lax
from jax.experimental import pallas as pl
from jax.experimental.pallas import tpu as pltpu
from jax.experimental.pallas import tpu_sc as plsc

N = 10000
E = 320000
DF = 128
DH = 64

NC = 2
NS = 16
NW = NC * NS
EPT = E // NW
CHUNK = 125
NCH = EPT // CHUNK
NPAD = 10240
RPS = NPAD // NS

_mesh = plsc.VectorSubcoreMesh(core_axis_name="c", subcore_axis_name="s")
_sc_params = pltpu.CompilerParams(use_tc_tiling_on_sc=False)


@functools.partial(
    pl.kernel,
    mesh=_mesh,
    out_type=jax.ShapeDtypeStruct((NC, NPAD, 8), jnp.float32),
    scratch_types=[
        pltpu.VMEM((NCH, CHUNK), jnp.int32),
        pltpu.VMEM((CHUNK, 8), jnp.float32),
        pltpu.VMEM_SHARED((NPAD, 8), jnp.float32),
    ],
    compiler_params=_sc_params,
)
def _deg_sc(dst_hbm, ones_hbm, zeros_hbm, out_hbm, idx_v, ones_v, acc_sh):
    c = lax.axis_index("c")
    s = lax.axis_index("s")
    wid = c * NS + s
    pltpu.sync_copy(dst_hbm.at[wid], idx_v)
    pltpu.sync_copy(ones_hbm, ones_v)
    base = pl.multiple_of(s * RPS, 8)
    pltpu.sync_copy(zeros_hbm.at[pl.ds(base, RPS)],
                    acc_sh.at[pl.ds(base, RPS)])
    plsc.subcore_barrier()

    def body(j, carry):
        pltpu.sync_copy(ones_v, acc_sh.at[idx_v.at[j]], add=True)
        return carry

    lax.fori_loop(0, NCH, body, 0)
    plsc.subcore_barrier()
    pltpu.sync_copy(acc_sh.at[pl.ds(base, RPS)],
                    out_hbm.at[c].at[pl.ds(base, RPS)])


def _make_agg(D):
    @functools.partial(
        pl.kernel,
        mesh=_mesh,
        out_type=jax.ShapeDtypeStruct((NC, NPAD, D), jnp.float32),
        scratch_types=[
            pltpu.VMEM((NCH, CHUNK), jnp.int32),
            pltpu.VMEM((NCH, CHUNK), jnp.int32),
            [pltpu.VMEM((CHUNK, D), jnp.float32) for _ in range(4)],
            pltpu.VMEM_SHARED((NPAD, D), jnp.float32),
            [pltpu.SemaphoreType.DMA for _ in range(4)],
        ],
        compiler_params=_sc_params,
    )
    def agg(src_hbm, dst_hbm, g_hbm, zeros_hbm, out_hbm,
            src_v, dst_v, rows, acc_sh, sems):
        c = lax.axis_index("c")
        s = lax.axis_index("s")
        wid = c * NS + s
        pltpu.sync_copy(src_hbm.at[wid], src_v)
        pltpu.sync_copy(dst_hbm.at[wid], dst_v)
        for k in range(3):
            pltpu.async_copy(g_hbm.at[src_v.at[k]], rows[k], sems[k])
        base = pl.multiple_of(s * RPS, 8)
        pltpu.sync_copy(zeros_hbm.at[pl.ds(base, RPS)],
                        acc_sh.at[pl.ds(base, RPS)])
        plsc.subcore_barrier()

        def body(h, carry):
            j = h * 4
            for k in range(4):
                nxt = j + k + 3

                @pl.when(nxt < NCH)
                def _(k=k, nxt=nxt):
                    pltpu.async_copy(g_hbm.at[src_v.at[nxt]],
                                     rows[(k + 3) % 4], sems[(k + 3) % 4])

                pltpu.make_async_copy(g_hbm.at[src_v.at[j + k]], rows[k],
                                      sems[k]).wait()
                pltpu.sync_copy(rows[k], acc_sh.at[dst_v.at[j + k]], add=True)
            return carry

        lax.fori_loop(0, NCH // 4, body, 0)
        plsc.subcore_barrier()
        pltpu.sync_copy(acc_sh.at[pl.ds(base, RPS)],
                        out_hbm.at[c].at[pl.ds(base, RPS)])

    return agg


_agg64 = _make_agg(DH)


_BLK = 2000
_MM = (((1,), (0,)), ((), ()))


def _dinv_of(dp_ref):
    deg = dp_ref[0, :, 0:1] + dp_ref[1, :, 0:1] + 1.0
    return lax.rsqrt(deg)


def _stage_a(x, W1, degP):
    def body(x_ref, w_ref, dp_ref, o_ref):
        dinv = _dinv_of(dp_ref)
        t = lax.dot_general(x_ref[...], w_ref[...], _MM,
                            preferred_element_type=jnp.float32)
        o_ref[...] = dinv * t

    return pl.pallas_call(
        body,
        grid=(N // _BLK,),
        in_specs=[
            pl.BlockSpec((_BLK, DF), lambda i: (i, 0)),
            pl.BlockSpec((DF, DH), lambda i: (0, 0)),
            pl.BlockSpec((NC, _BLK, 8), lambda i: (0, i, 0)),
        ],
        out_specs=pl.BlockSpec((_BLK, DH), lambda i: (i, 0)),
        out_shape=jax.ShapeDtypeStruct((N, DH), jnp.float32),
    )(x, W1, degP)


def _stage_b(P, g, degP, b, W):
    def body(p_ref, g_ref, dp_ref, b_ref, w_ref, o_ref):
        dinv = _dinv_of(dp_ref)
        h = jnp.maximum(dinv * (p_ref[0] + p_ref[1] + g_ref[...]) + b_ref[...],
                        0.0)
        o_ref[...] = dinv * lax.dot_general(
            h, w_ref[...], _MM, preferred_element_type=jnp.float32)

    return pl.pallas_call(
        body,
        grid=(N // _BLK,),
        in_specs=[
            pl.BlockSpec((NC, _BLK, DH), lambda i: (0, i, 0)),
            pl.BlockSpec((_BLK, DH), lambda i: (i, 0)),
            pl.BlockSpec((NC, _BLK, 8), lambda i: (0, i, 0)),
            pl.BlockSpec((1, DH), lambda i: (0, 0)),
            pl.BlockSpec((DH, DH), lambda i: (0, 0)),
        ],
        out_specs=pl.BlockSpec((_BLK, DH), lambda i: (i, 0)),
        out_shape=jax.ShapeDtypeStruct((N, DH), jnp.float32),
    )(P, g, degP, b, W)


def _stage_c(P, g, degP, b, W3, W5):
    def body(p_ref, g_ref, dp_ref, b_ref, w3_ref, w5_ref, o3_ref, o5_ref):
        dinv = _dinv_of(dp_ref)
        h = jnp.maximum(dinv * (p_ref[0] + p_ref[1] + g_ref[...]) + b_ref[...],
                        0.0)
        o3_ref[...] = dinv * lax.dot_general(
            h, w3_ref[...], _MM, preferred_element_type=jnp.float32)
        o5_ref[...] = dinv * lax.dot_general(
            h, w5_ref[...], _MM, preferred_element_type=jnp.float32)

    return pl.pallas_call(
        body,
        grid=(N // _BLK,),
        in_specs=[
            pl.BlockSpec((NC, _BLK, DH), lambda i: (0, i, 0)),
            pl.BlockSpec((_BLK, DH), lambda i: (i, 0)),
            pl.BlockSpec((NC, _BLK, 8), lambda i: (0, i, 0)),
            pl.BlockSpec((1, DH), lambda i: (0, 0)),
            pl.BlockSpec((DH, DH), lambda i: (0, 0)),
            pl.BlockSpec((DH, DH), lambda i: (0, 0)),
        ],
        out_specs=[
            pl.BlockSpec((_BLK, DH), lambda i: (i, 0)),
            pl.BlockSpec((_BLK, DH), lambda i: (i, 0)),
        ],
        out_shape=[
            jax.ShapeDtypeStruct((N, DH), jnp.float32),
            jax.ShapeDtypeStruct((N, DH), jnp.float32),
        ],
    )(P, g, degP, b, W3, W5)


def _stage_d(P, g3, degP, b):
    def body(p_ref, g3_ref, dp_ref, b_ref, o_ref):
        dinv = _dinv_of(dp_ref)
        xa = jnp.maximum(
            dinv * (p_ref[0] + p_ref[1] + g3_ref[...]) + b_ref[...], 0.0)
        o_ref[...] = dinv * xa

    return pl.pallas_call(
        body,
        grid=(N // _BLK,),
        in_specs=[
            pl.BlockSpec((NC, _BLK, DH), lambda i: (0, i, 0)),
            pl.BlockSpec((_BLK, DH), lambda i: (i, 0)),
            pl.BlockSpec((NC, _BLK, 8), lambda i: (0, i, 0)),
            pl.BlockSpec((1, DH), lambda i: (0, 0)),
        ],
        out_specs=pl.BlockSpec((_BLK, DH), lambda i: (i, 0)),
        out_shape=jax.ShapeDtypeStruct((N, DH), jnp.float32),
    )(P, g3, degP, b)


def _stage_e(P4, g4, P5, g5, degP, W4, b4, b5):
    def body(p4_ref, g4_ref, p5_ref, g5_ref, dp_ref, w4_ref, b4_ref, b5_ref,
             xh_ref, s_ref):
        dinv = _dinv_of(dp_ref)
        a4 = dinv * (p4_ref[0] + p4_ref[1] + g4_ref[...])
        xh_ref[...] = jnp.maximum(
            lax.dot_general(a4, w4_ref[...], _MM,
                            preferred_element_type=jnp.float32) + b4_ref[...],
            0.0)
        s_ref[...] = jnp.maximum(
            dinv * (p5_ref[0] + p5_ref[1] + g5_ref[...]) + b5_ref[...], 0.0)

    return pl.pallas_call(
        body,
        grid=(N // _BLK,),
        in_specs=[
            pl.BlockSpec((NC, _BLK, DH), lambda i: (0, i, 0)),
            pl.BlockSpec((_BLK, DH), lambda i: (i, 0)),
            pl.BlockSpec((NC, _BLK, DH), lambda i: (0, i, 0)),
            pl.BlockSpec((_BLK, DH), lambda i: (i, 0)),
            pl.BlockSpec((NC, _BLK, 8), lambda i: (0, i, 0)),
            pl.BlockSpec((DH, DF), lambda i: (0, 0)),
            pl.BlockSpec((1, DF), lambda i: (0, 0)),
            pl.BlockSpec((1, DH), lambda i: (0, 0)),
        ],
        out_specs=[
            pl.BlockSpec((_BLK, DF), lambda i: (i, 0)),
            pl.BlockSpec((_BLK, DH), lambda i: (i, 0)),
        ],
        out_shape=[
            jax.ShapeDtypeStruct((N, DF), jnp.float32),
            jax.ShapeDtypeStruct((N, DH), jnp.float32),
        ],
    )(P4, g4, P5, g5, degP, W4, b4, b5)


def _stage_f(s):
    BM, BN = 1000, 1024
    _TT = (((1,), (1,)), ((), ()))

    def body(a_ref, b_ref, o_ref):
        o_ref[...] = lax.dot_general(a_ref[...], b_ref[...], _TT,
                                     preferred_element_type=jnp.float32)

    return pl.pallas_call(
        body,
        grid=(N // BM, pl.cdiv(N, BN)),
        in_specs=[
            pl.BlockSpec((BM, DH), lambda i, j: (i, 0)),
            pl.BlockSpec((BN, DH), lambda i, j: (j, 0)),
        ],
        out_specs=pl.BlockSpec((BM, BN), lambda i, j: (i, j)),
        out_shape=jax.ShapeDtypeStruct((N, N), jnp.float32),
        compiler_params=pltpu.CompilerParams(
            dimension_semantics=("parallel", "parallel")),
    )(s, s)


def kernel(x, edge_index, W1, b1, W2, b2, W3, b3, W4, b4, W5, b5):
    src = edge_index[0].astype(jnp.int32).reshape(NW, NCH, CHUNK)
    dst = edge_index[1].astype(jnp.int32).reshape(NW, NCH, CHUNK)
    ones8 = jnp.ones((CHUNK, 8), jnp.float32)
    z8 = jnp.zeros((NPAD, 8), jnp.float32)
    z64 = jnp.zeros((NPAD, DH), jnp.float32)

    degP = _deg_sc(dst, ones8, z8)
    g1 = _stage_a(x, W1, degP)
    P1 = _agg64(src, dst, g1, z64)
    g2 = _stage_b(P1, g1, degP, b1.reshape(1, -1), W2)
    P2 = _agg64(src, dst, g2, z64)
    g3, g5 = _stage_c(P2, g2, degP, b2.reshape(1, -1), W3, W5)
    P3 = _agg64(src, dst, g3, z64)
    g4 = _stage_d(P3, g3, degP, b3.reshape(1, -1))
    P4 = _agg64(src, dst, g4, z64)
    P5 = _agg64(src, dst, g5, z64)
    X_hat, s = _stage_e(P4, g4, P5, g5, degP, W4, b4.reshape(1, -1),
                        b5.reshape(1, -1))
    A_hat = _stage_f(s)
    return (A_hat, X_hat)


# speedup vs baseline: 27.4317x; 1.0225x over previous
import functools

import jax
import jax.numpy as jnp
from jax import lax
from jax.experimental import pallas as pl
from jax.experimental.pallas import tpu as pltpu
from jax.experimental.pallas import tpu_sc as plsc

N = 10000
E = 320000
DF = 128
DH = 64

NC = 2
NS = 16
NW = NC * NS
EPT = E // NW
CHUNK = 125
NCH = EPT // CHUNK
NPAD = 10240
RPS = NPAD // NS

_mesh = plsc.VectorSubcoreMesh(core_axis_name="c", subcore_axis_name="s")
_sc_params = pltpu.CompilerParams(use_tc_tiling_on_sc=False)


@functools.partial(
    pl.kernel,
    mesh=_mesh,
    out_type=jax.ShapeDtypeStruct((NC, NPAD, 8), jnp.float32),
    scratch_types=[
        pltpu.VMEM((NCH, CHUNK), jnp.int32),
        pltpu.VMEM((CHUNK, 8), jnp.float32),
        pltpu.VMEM_SHARED((NPAD, 8), jnp.float32),
    ],
    compiler_params=_sc_params,
)
def _deg_sc(dst_hbm, ones_hbm, zeros_hbm, out_hbm, idx_v, ones_v, acc_sh):
    c = lax.axis_index("c")
    s = lax.axis_index("s")
    wid = c * NS + s
    pltpu.sync_copy(dst_hbm.at[wid], idx_v)
    pltpu.sync_copy(ones_hbm, ones_v)
    base = pl.multiple_of(s * RPS, 8)
    pltpu.sync_copy(zeros_hbm.at[pl.ds(base, RPS)],
                    acc_sh.at[pl.ds(base, RPS)])
    plsc.subcore_barrier()

    def body(j, carry):
        pltpu.sync_copy(ones_v, acc_sh.at[idx_v.at[j]], add=True)
        return carry

    lax.fori_loop(0, NCH, body, 0)
    plsc.subcore_barrier()
    pltpu.sync_copy(acc_sh.at[pl.ds(base, RPS)],
                    out_hbm.at[c].at[pl.ds(base, RPS)])


def _make_agg(D):
    @functools.partial(
        pl.kernel,
        mesh=_mesh,
        out_type=jax.ShapeDtypeStruct((NC, NPAD, D), jnp.float32),
        scratch_types=[
            pltpu.VMEM((NCH, CHUNK), jnp.int32),
            pltpu.VMEM((NCH, CHUNK), jnp.int32),
            [pltpu.VMEM((CHUNK, D), jnp.float32) for _ in range(4)],
            pltpu.VMEM_SHARED((NPAD, D), jnp.float32),
            [pltpu.SemaphoreType.DMA for _ in range(4)],
        ],
        compiler_params=_sc_params,
    )
    def agg(src_hbm, dst_hbm, g_hbm, zeros_hbm, out_hbm,
            src_v, dst_v, rows, acc_sh, sems):
        c = lax.axis_index("c")
        s = lax.axis_index("s")
        wid = c * NS + s
        pltpu.sync_copy(src_hbm.at[wid], src_v)
        pltpu.sync_copy(dst_hbm.at[wid], dst_v)
        for k in range(3):
            pltpu.async_copy(g_hbm.at[src_v.at[k]], rows[k], sems[k])
        base = pl.multiple_of(s * RPS, 8)
        pltpu.sync_copy(zeros_hbm.at[pl.ds(base, RPS)],
                        acc_sh.at[pl.ds(base, RPS)])
        plsc.subcore_barrier()

        def body(h, carry):
            j = h * 4
            for k in range(4):
                nxt = j + k + 3

                @pl.when(nxt < NCH)
                def _(k=k, nxt=nxt):
                    pltpu.async_copy(g_hbm.at[src_v.at[nxt]],
                                     rows[(k + 3) % 4], sems[(k + 3) % 4])

                pltpu.make_async_copy(g_hbm.at[src_v.at[j + k]], rows[k],
                                      sems[k]).wait()
                pltpu.sync_copy(rows[k], acc_sh.at[dst_v.at[j + k]], add=True)
            return carry

        lax.fori_loop(0, NCH // 4, body, 0)
        plsc.subcore_barrier()
        pltpu.sync_copy(acc_sh.at[pl.ds(base, RPS)],
                        out_hbm.at[c].at[pl.ds(base, RPS)])

    return agg


_agg64 = _make_agg(DH)


_BLK = 2000
_MM = (((1,), (0,)), ((), ()))


def _dinv_of(dp_ref):
    deg = dp_ref[0, :, 0:1] + dp_ref[1, :, 0:1] + 1.0
    return lax.rsqrt(deg)


def _stage_a1(x, W1):
    def body(x_ref, w_ref, o_ref):
        o_ref[...] = lax.dot_general(x_ref[...], w_ref[...], _MM,
                                     preferred_element_type=jnp.float32)

    return pl.pallas_call(
        body,
        grid=(N // _BLK,),
        in_specs=[
            pl.BlockSpec((_BLK, DF), lambda i: (i, 0)),
            pl.BlockSpec((DF, DH), lambda i: (0, 0)),
        ],
        out_specs=pl.BlockSpec((_BLK, DH), lambda i: (i, 0)),
        out_shape=jax.ShapeDtypeStruct((N, DH), jnp.float32),
    )(x, W1)


def _stage_a2(t, degP):
    def body(t_ref, dp_ref, o_ref):
        o_ref[...] = _dinv_of(dp_ref) * t_ref[...]

    return pl.pallas_call(
        body,
        grid=(N // _BLK,),
        in_specs=[
            pl.BlockSpec((_BLK, DH), lambda i: (i, 0)),
            pl.BlockSpec((NC, _BLK, 8), lambda i: (0, i, 0)),
        ],
        out_specs=pl.BlockSpec((_BLK, DH), lambda i: (i, 0)),
        out_shape=jax.ShapeDtypeStruct((N, DH), jnp.float32),
    )(t, degP)


def _stage_b(P, g, degP, b, W):
    def body(p_ref, g_ref, dp_ref, b_ref, w_ref, o_ref):
        dinv = _dinv_of(dp_ref)
        h = jnp.maximum(dinv * (p_ref[0] + p_ref[1] + g_ref[...]) + b_ref[...],
                        0.0)
        o_ref[...] = dinv * lax.dot_general(
            h, w_ref[...], _MM, preferred_element_type=jnp.float32)

    return pl.pallas_call(
        body,
        grid=(N // _BLK,),
        in_specs=[
            pl.BlockSpec((NC, _BLK, DH), lambda i: (0, i, 0)),
            pl.BlockSpec((_BLK, DH), lambda i: (i, 0)),
            pl.BlockSpec((NC, _BLK, 8), lambda i: (0, i, 0)),
            pl.BlockSpec((1, DH), lambda i: (0, 0)),
            pl.BlockSpec((DH, DH), lambda i: (0, 0)),
        ],
        out_specs=pl.BlockSpec((_BLK, DH), lambda i: (i, 0)),
        out_shape=jax.ShapeDtypeStruct((N, DH), jnp.float32),
    )(P, g, degP, b, W)


def _stage_c(P, g, degP, b, W3, W5):
    def body(p_ref, g_ref, dp_ref, b_ref, w3_ref, w5_ref, o3_ref, o5_ref):
        dinv = _dinv_of(dp_ref)
        h = jnp.maximum(dinv * (p_ref[0] + p_ref[1] + g_ref[...]) + b_ref[...],
                        0.0)
        o3_ref[...] = dinv * lax.dot_general(
            h, w3_ref[...], _MM, preferred_element_type=jnp.float32)
        o5_ref[...] = dinv * lax.dot_general(
            h, w5_ref[...], _MM, preferred_element_type=jnp.float32)

    return pl.pallas_call(
        body,
        grid=(N // _BLK,),
        in_specs=[
            pl.BlockSpec((NC, _BLK, DH), lambda i: (0, i, 0)),
            pl.BlockSpec((_BLK, DH), lambda i: (i, 0)),
            pl.BlockSpec((NC, _BLK, 8), lambda i: (0, i, 0)),
            pl.BlockSpec((1, DH), lambda i: (0, 0)),
            pl.BlockSpec((DH, DH), lambda i: (0, 0)),
            pl.BlockSpec((DH, DH), lambda i: (0, 0)),
        ],
        out_specs=[
            pl.BlockSpec((_BLK, DH), lambda i: (i, 0)),
            pl.BlockSpec((_BLK, DH), lambda i: (i, 0)),
        ],
        out_shape=[
            jax.ShapeDtypeStruct((N, DH), jnp.float32),
            jax.ShapeDtypeStruct((N, DH), jnp.float32),
        ],
    )(P, g, degP, b, W3, W5)


def _stage_d(P, g3, degP, b):
    def body(p_ref, g3_ref, dp_ref, b_ref, o_ref):
        dinv = _dinv_of(dp_ref)
        xa = jnp.maximum(
            dinv * (p_ref[0] + p_ref[1] + g3_ref[...]) + b_ref[...], 0.0)
        o_ref[...] = dinv * xa

    return pl.pallas_call(
        body,
        grid=(N // _BLK,),
        in_specs=[
            pl.BlockSpec((NC, _BLK, DH), lambda i: (0, i, 0)),
            pl.BlockSpec((_BLK, DH), lambda i: (i, 0)),
            pl.BlockSpec((NC, _BLK, 8), lambda i: (0, i, 0)),
            pl.BlockSpec((1, DH), lambda i: (0, 0)),
        ],
        out_specs=pl.BlockSpec((_BLK, DH), lambda i: (i, 0)),
        out_shape=jax.ShapeDtypeStruct((N, DH), jnp.float32),
    )(P, g3, degP, b)


def _stage_e5(P5, g5, degP, b5):
    def body(p5_ref, g5_ref, dp_ref, b5_ref, s_ref):
        dinv = _dinv_of(dp_ref)
        s_ref[...] = jnp.maximum(
            dinv * (p5_ref[0] + p5_ref[1] + g5_ref[...]) + b5_ref[...], 0.0)

    return pl.pallas_call(
        body,
        grid=(N // _BLK,),
        in_specs=[
            pl.BlockSpec((NC, _BLK, DH), lambda i: (0, i, 0)),
            pl.BlockSpec((_BLK, DH), lambda i: (i, 0)),
            pl.BlockSpec((NC, _BLK, 8), lambda i: (0, i, 0)),
            pl.BlockSpec((1, DH), lambda i: (0, 0)),
        ],
        out_specs=pl.BlockSpec((_BLK, DH), lambda i: (i, 0)),
        out_shape=jax.ShapeDtypeStruct((N, DH), jnp.float32),
    )(P5, g5, degP, b5)


def _stage_e4(P4, g4, degP, W4, b4):
    def body(p4_ref, g4_ref, dp_ref, w4_ref, b4_ref, xh_ref):
        dinv = _dinv_of(dp_ref)
        a4 = dinv * (p4_ref[0] + p4_ref[1] + g4_ref[...])
        xh_ref[...] = jnp.maximum(
            lax.dot_general(a4, w4_ref[...], _MM,
                            preferred_element_type=jnp.float32) + b4_ref[...],
            0.0)

    return pl.pallas_call(
        body,
        grid=(N // _BLK,),
        in_specs=[
            pl.BlockSpec((NC, _BLK, DH), lambda i: (0, i, 0)),
            pl.BlockSpec((_BLK, DH), lambda i: (i, 0)),
            pl.BlockSpec((NC, _BLK, 8), lambda i: (0, i, 0)),
            pl.BlockSpec((DH, DF), lambda i: (0, 0)),
            pl.BlockSpec((1, DF), lambda i: (0, 0)),
        ],
        out_specs=pl.BlockSpec((_BLK, DF), lambda i: (i, 0)),
        out_shape=jax.ShapeDtypeStruct((N, DF), jnp.float32),
    )(P4, g4, degP, W4, b4)


def _stage_f(s):
    BM, BN = 1000, 1024
    _TT = (((1,), (1,)), ((), ()))

    def body(a_ref, b_ref, o_ref):
        o_ref[...] = lax.dot_general(a_ref[...], b_ref[...], _TT,
                                     preferred_element_type=jnp.float32)

    return pl.pallas_call(
        body,
        grid=(N // BM, pl.cdiv(N, BN)),
        in_specs=[
            pl.BlockSpec((BM, DH), lambda i, j: (i, 0)),
            pl.BlockSpec((BN, DH), lambda i, j: (j, 0)),
        ],
        out_specs=pl.BlockSpec((BM, BN), lambda i, j: (i, j)),
        out_shape=jax.ShapeDtypeStruct((N, N), jnp.float32),
        compiler_params=pltpu.CompilerParams(
            dimension_semantics=("parallel", "parallel")),
    )(s, s)


def kernel(x, edge_index, W1, b1, W2, b2, W3, b3, W4, b4, W5, b5):
    src = edge_index[0].astype(jnp.int32).reshape(NW, NCH, CHUNK)
    dst = edge_index[1].astype(jnp.int32).reshape(NW, NCH, CHUNK)
    ones8 = jnp.ones((CHUNK, 8), jnp.float32)
    z8 = jnp.zeros((NPAD, 8), jnp.float32)
    z64 = jnp.zeros((NPAD, DH), jnp.float32)

    degP = _deg_sc(dst, ones8, z8)
    t1 = _stage_a1(x, W1)
    g1 = _stage_a2(t1, degP)
    P1 = _agg64(src, dst, g1, z64)
    g2 = _stage_b(P1, g1, degP, b1.reshape(1, -1), W2)
    P2 = _agg64(src, dst, g2, z64)
    g3, g5 = _stage_c(P2, g2, degP, b2.reshape(1, -1), W3, W5)
    P3 = _agg64(src, dst, g3, z64)
    P5 = _agg64(src, dst, g5, z64)
    g4 = _stage_d(P3, g3, degP, b3.reshape(1, -1))
    s = _stage_e5(P5, g5, degP, b5.reshape(1, -1))
    P4 = _agg64(src, dst, g4, z64)
    A_hat = _stage_f(s)
    X_hat = _stage_e4(P4, g4, degP, W4, b4.reshape(1, -1))
    return (A_hat, X_hat)


# speedup vs baseline: 29.0475x; 1.0589x over previous
import functools

import jax
import jax.numpy as jnp
from jax import lax
from jax.experimental import pallas as pl
from jax.experimental.pallas import tpu as pltpu
from jax.experimental.pallas import tpu_sc as plsc

N = 10000
E = 320000
DF = 128
DH = 64

NC = 2
NS = 16
NW = NC * NS
EPT = E // NW
CHUNK = 125
NCH = EPT // CHUNK
NPAD = 10240
RPS = NPAD // NS

_mesh = plsc.VectorSubcoreMesh(core_axis_name="c", subcore_axis_name="s")
_sc_params = pltpu.CompilerParams(use_tc_tiling_on_sc=False)


@functools.partial(
    pl.kernel,
    mesh=_mesh,
    out_type=jax.ShapeDtypeStruct((NC, NPAD, 8), jnp.float32),
    scratch_types=[
        pltpu.VMEM((NCH, CHUNK), jnp.int32),
        pltpu.VMEM((CHUNK, 8), jnp.float32),
        pltpu.VMEM_SHARED((NPAD, 8), jnp.float32),
    ],
    compiler_params=_sc_params,
)
def _deg_sc(dst_hbm, ones_hbm, zeros_hbm, out_hbm, idx_v, ones_v, acc_sh):
    c = lax.axis_index("c")
    s = lax.axis_index("s")
    wid = c * NS + s
    pltpu.sync_copy(dst_hbm.at[wid], idx_v)
    pltpu.sync_copy(ones_hbm, ones_v)
    base = pl.multiple_of(s * RPS, 8)
    pltpu.sync_copy(zeros_hbm.at[pl.ds(base, RPS)],
                    acc_sh.at[pl.ds(base, RPS)])
    plsc.subcore_barrier()

    def body(j, carry):
        pltpu.sync_copy(ones_v, acc_sh.at[idx_v.at[j]], add=True)
        return carry

    lax.fori_loop(0, NCH, body, 0)
    plsc.subcore_barrier()
    pltpu.sync_copy(acc_sh.at[pl.ds(base, RPS)],
                    out_hbm.at[c].at[pl.ds(base, RPS)])


def _make_agg(D):
    @functools.partial(
        pl.kernel,
        mesh=_mesh,
        out_type=jax.ShapeDtypeStruct((NC, NPAD, D), jnp.float32),
        scratch_types=[
            pltpu.VMEM((NCH, CHUNK), jnp.int32),
            pltpu.VMEM((NCH, CHUNK), jnp.int32),
            [pltpu.VMEM((CHUNK, D), jnp.float32) for _ in range(4)],
            pltpu.VMEM_SHARED((NPAD, D), jnp.float32),
            [pltpu.SemaphoreType.DMA for _ in range(4)],
        ],
        compiler_params=_sc_params,
    )
    def agg(src_hbm, dst_hbm, g_hbm, zeros_hbm, out_hbm,
            src_v, dst_v, rows, acc_sh, sems):
        c = lax.axis_index("c")
        s = lax.axis_index("s")
        wid = c * NS + s
        pltpu.sync_copy(src_hbm.at[wid], src_v)
        pltpu.sync_copy(dst_hbm.at[wid], dst_v)
        for k in range(3):
            pltpu.async_copy(g_hbm.at[src_v.at[k]], rows[k], sems[k])
        base = pl.multiple_of(s * RPS, 8)
        pltpu.sync_copy(zeros_hbm.at[pl.ds(base, RPS)],
                        acc_sh.at[pl.ds(base, RPS)])
        plsc.subcore_barrier()

        def body(h, carry):
            j = h * 4
            for k in range(4):
                nxt = j + k + 3

                @pl.when(nxt < NCH)
                def _(k=k, nxt=nxt):
                    pltpu.async_copy(g_hbm.at[src_v.at[nxt]],
                                     rows[(k + 3) % 4], sems[(k + 3) % 4])

                pltpu.make_async_copy(g_hbm.at[src_v.at[j + k]], rows[k],
                                      sems[k]).wait()
                pltpu.sync_copy(rows[k], acc_sh.at[dst_v.at[j + k]], add=True)
            return carry

        lax.fori_loop(0, NCH // 4, body, 0)
        plsc.subcore_barrier()
        pltpu.sync_copy(acc_sh.at[pl.ds(base, RPS)],
                        out_hbm.at[c].at[pl.ds(base, RPS)])

    return agg


_agg64 = _make_agg(DH)


_BLK = 2000
_MM = (((1,), (0,)), ((), ()))


def _dinv_of(dp_ref):
    deg = dp_ref[0, :, 0:1] + dp_ref[1, :, 0:1] + 1.0
    return lax.rsqrt(deg)


def _stage_a1(x, W1):
    def body(x_ref, w_ref, o_ref):
        o_ref[...] = lax.dot_general(x_ref[...], w_ref[...], _MM,
                                     preferred_element_type=jnp.float32)

    return pl.pallas_call(
        body,
        grid=(N // _BLK,),
        in_specs=[
            pl.BlockSpec((_BLK, DF), lambda i: (i, 0)),
            pl.BlockSpec((DF, DH), lambda i: (0, 0)),
        ],
        out_specs=pl.BlockSpec((_BLK, DH), lambda i: (i, 0)),
        out_shape=jax.ShapeDtypeStruct((N, DH), jnp.float32),
    )(x, W1)


def _stage_a2(t, degP):
    def body(t_ref, dp_ref, o_ref):
        o_ref[...] = _dinv_of(dp_ref) * t_ref[...]

    return pl.pallas_call(
        body,
        grid=(N // _BLK,),
        in_specs=[
            pl.BlockSpec((_BLK, DH), lambda i: (i, 0)),
            pl.BlockSpec((NC, _BLK, 8), lambda i: (0, i, 0)),
        ],
        out_specs=pl.BlockSpec((_BLK, DH), lambda i: (i, 0)),
        out_shape=jax.ShapeDtypeStruct((N, DH), jnp.float32),
    )(t, degP)


def _stage_b(P, g, degP, b, W):
    def body(p_ref, g_ref, dp_ref, b_ref, w_ref, o_ref):
        dinv = _dinv_of(dp_ref)
        h = jnp.maximum(dinv * (p_ref[0] + p_ref[1] + g_ref[...]) + b_ref[...],
                        0.0)
        o_ref[...] = dinv * lax.dot_general(
            h, w_ref[...], _MM, preferred_element_type=jnp.float32)

    return pl.pallas_call(
        body,
        grid=(N // _BLK,),
        in_specs=[
            pl.BlockSpec((NC, _BLK, DH), lambda i: (0, i, 0)),
            pl.BlockSpec((_BLK, DH), lambda i: (i, 0)),
            pl.BlockSpec((NC, _BLK, 8), lambda i: (0, i, 0)),
            pl.BlockSpec((1, DH), lambda i: (0, 0)),
            pl.BlockSpec((DH, DH), lambda i: (0, 0)),
        ],
        out_specs=pl.BlockSpec((_BLK, DH), lambda i: (i, 0)),
        out_shape=jax.ShapeDtypeStruct((N, DH), jnp.float32),
    )(P, g, degP, b, W)


def _stage_c(P, g, degP, b, W3, W5):
    def body(p_ref, g_ref, dp_ref, b_ref, w3_ref, w5_ref, o3_ref, o5_ref):
        dinv = _dinv_of(dp_ref)
        h = jnp.maximum(dinv * (p_ref[0] + p_ref[1] + g_ref[...]) + b_ref[...],
                        0.0)
        o3_ref[...] = dinv * lax.dot_general(
            h, w3_ref[...], _MM, preferred_element_type=jnp.float32)
        o5_ref[...] = dinv * lax.dot_general(
            h, w5_ref[...], _MM, preferred_element_type=jnp.float32)

    return pl.pallas_call(
        body,
        grid=(N // _BLK,),
        in_specs=[
            pl.BlockSpec((NC, _BLK, DH), lambda i: (0, i, 0)),
            pl.BlockSpec((_BLK, DH), lambda i: (i, 0)),
            pl.BlockSpec((NC, _BLK, 8), lambda i: (0, i, 0)),
            pl.BlockSpec((1, DH), lambda i: (0, 0)),
            pl.BlockSpec((DH, DH), lambda i: (0, 0)),
            pl.BlockSpec((DH, DH), lambda i: (0, 0)),
        ],
        out_specs=[
            pl.BlockSpec((_BLK, DH), lambda i: (i, 0)),
            pl.BlockSpec((_BLK, DH), lambda i: (i, 0)),
        ],
        out_shape=[
            jax.ShapeDtypeStruct((N, DH), jnp.float32),
            jax.ShapeDtypeStruct((N, DH), jnp.float32),
        ],
    )(P, g, degP, b, W3, W5)


def _stage_d(P, g3, degP, b):
    def body(p_ref, g3_ref, dp_ref, b_ref, o_ref):
        dinv = _dinv_of(dp_ref)
        xa = jnp.maximum(
            dinv * (p_ref[0] + p_ref[1] + g3_ref[...]) + b_ref[...], 0.0)
        o_ref[...] = dinv * xa

    return pl.pallas_call(
        body,
        grid=(N // _BLK,),
        in_specs=[
            pl.BlockSpec((NC, _BLK, DH), lambda i: (0, i, 0)),
            pl.BlockSpec((_BLK, DH), lambda i: (i, 0)),
            pl.BlockSpec((NC, _BLK, 8), lambda i: (0, i, 0)),
            pl.BlockSpec((1, DH), lambda i: (0, 0)),
        ],
        out_specs=pl.BlockSpec((_BLK, DH), lambda i: (i, 0)),
        out_shape=jax.ShapeDtypeStruct((N, DH), jnp.float32),
    )(P, g3, degP, b)


def _stage_e5(P5, g5, degP, b5):
    def body(p5_ref, g5_ref, dp_ref, b5_ref, s_ref):
        dinv = _dinv_of(dp_ref)
        s_ref[...] = jnp.maximum(
            dinv * (p5_ref[0] + p5_ref[1] + g5_ref[...]) + b5_ref[...], 0.0)

    return pl.pallas_call(
        body,
        grid=(N // _BLK,),
        in_specs=[
            pl.BlockSpec((NC, _BLK, DH), lambda i: (0, i, 0)),
            pl.BlockSpec((_BLK, DH), lambda i: (i, 0)),
            pl.BlockSpec((NC, _BLK, 8), lambda i: (0, i, 0)),
            pl.BlockSpec((1, DH), lambda i: (0, 0)),
        ],
        out_specs=pl.BlockSpec((_BLK, DH), lambda i: (i, 0)),
        out_shape=jax.ShapeDtypeStruct((N, DH), jnp.float32),
    )(P5, g5, degP, b5)


def _stage_e4(P4, g4, degP, W4, b4):
    def body(p4_ref, g4_ref, dp_ref, w4_ref, b4_ref, xh_ref):
        dinv = _dinv_of(dp_ref)
        a4 = dinv * (p4_ref[0] + p4_ref[1] + g4_ref[...])
        xh_ref[...] = jnp.maximum(
            lax.dot_general(a4, w4_ref[...], _MM,
                            preferred_element_type=jnp.float32) + b4_ref[...],
            0.0)

    return pl.pallas_call(
        body,
        grid=(N // _BLK,),
        in_specs=[
            pl.BlockSpec((NC, _BLK, DH), lambda i: (0, i, 0)),
            pl.BlockSpec((_BLK, DH), lambda i: (i, 0)),
            pl.BlockSpec((NC, _BLK, 8), lambda i: (0, i, 0)),
            pl.BlockSpec((DH, DF), lambda i: (0, 0)),
            pl.BlockSpec((1, DF), lambda i: (0, 0)),
        ],
        out_specs=pl.BlockSpec((_BLK, DF), lambda i: (i, 0)),
        out_shape=jax.ShapeDtypeStruct((N, DF), jnp.float32),
    )(P4, g4, degP, W4, b4)


def _stage_f(s):
    BM, BN = 2000, 2048
    _TT = (((1,), (1,)), ((), ()))

    def body(a_ref, b_ref, o_ref):
        o_ref[...] = lax.dot_general(a_ref[...], b_ref[...], _TT,
                                     preferred_element_type=jnp.float32)

    return pl.pallas_call(
        body,
        grid=(N // BM, pl.cdiv(N, BN)),
        in_specs=[
            pl.BlockSpec((BM, DH), lambda i, j: (i, 0)),
            pl.BlockSpec((BN, DH), lambda i, j: (j, 0)),
        ],
        out_specs=pl.BlockSpec((BM, BN), lambda i, j: (i, j)),
        out_shape=jax.ShapeDtypeStruct((N, N), jnp.float32),
        compiler_params=pltpu.CompilerParams(
            dimension_semantics=("parallel", "parallel")),
    )(s, s)


def kernel(x, edge_index, W1, b1, W2, b2, W3, b3, W4, b4, W5, b5):
    src = edge_index[0].astype(jnp.int32).reshape(NW, NCH, CHUNK)
    dst = edge_index[1].astype(jnp.int32).reshape(NW, NCH, CHUNK)
    ones8 = jnp.ones((CHUNK, 8), jnp.float32)
    z8 = jnp.zeros((NPAD, 8), jnp.float32)
    z64 = jnp.zeros((NPAD, DH), jnp.float32)

    degP = _deg_sc(dst, ones8, z8)
    t1 = _stage_a1(x, W1)
    g1 = _stage_a2(t1, degP)
    P1 = _agg64(src, dst, g1, z64)
    g2 = _stage_b(P1, g1, degP, b1.reshape(1, -1), W2)
    P2 = _agg64(src, dst, g2, z64)
    g3, g5 = _stage_c(P2, g2, degP, b2.reshape(1, -1), W3, W5)
    P3 = _agg64(src, dst, g3, z64)
    P5 = _agg64(src, dst, g5, z64)
    g4 = _stage_d(P3, g3, degP, b3.reshape(1, -1))
    s = _stage_e5(P5, g5, degP, b5.reshape(1, -1))
    P4 = _agg64(src, dst, g4, z64)
    A_hat = _stage_f(s)
    X_hat = _stage_e4(P4, g4, degP, W4, b4.reshape(1, -1))
    return (A_hat, X_hat)
